# baseline (speedup 1.0000x reference)
import functools

import jax
import jax.numpy as jnp
from jax import lax
from jax.experimental import pallas as pl
from jax.experimental.pallas import tpu as pltpu
from jax.experimental.pallas import tpu_sc as plsc

EMB_FREQS = 32
MAX_POS = 10000.0
BLK = 8192
SUB = 512
QWIN = 128
NCHUNK = 1
CORE0_FRAC = 0.25


def _sin_embed(p):
    k = jnp.arange(EMB_FREQS, dtype=jnp.float32)
    freqs = 1.0 / (MAX_POS ** (k / EMB_FREQS))
    ang = p[..., None] * freqs
    emb = jnp.concatenate([jnp.sin(ang), jnp.cos(ang)], axis=-1)
    return emb.reshape(p.shape[0], -1)


def _gelu(v):
    return 0.5 * v * (1.0 + lax.erf(v * 0.7071067811865476))


def _axis_table_kernel(t_ref, fvec_ref, offv_ref, w_ref, o_ref):
    ra = t_ref.shape[0]
    emb = fvec_ref.shape[1]
    r = ra // 3
    ang = t_ref[...] * fvec_ref[...] + offv_ref[...]
    e = jnp.sin(ang)
    grp = lax.broadcasted_iota(jnp.int32, (ra, emb), 0) // r
    colgrp = lax.broadcasted_iota(jnp.int32, (ra, emb), 1) // (emb // 3)
    e = jnp.where(grp == colgrp, e, 0.0)
    o_ref[...] = jnp.dot(e, w_ref[...], preferred_element_type=jnp.float32)


def _axis_table(t, fvec, offv, w):
    ra = t.shape[0]
    emb = fvec.shape[1]
    m = w.shape[1]
    return pl.pallas_call(
        _axis_table_kernel,
        out_shape=jax.ShapeDtypeStruct((ra, m), jnp.float32),
    )(t, fvec, offv, w)


def _build_a_kernel(tab_ref, f_ref, wfeat_ref, o_ref):
    n = o_ref.shape[0]
    ra = tab_ref.shape[0]
    r = ra // 3
    ri = lax.broadcasted_iota(jnp.int32, (n, ra), 0)
    ji = lax.broadcasted_iota(jnp.int32, (n, ra), 1)
    sel = ((ji == (ri // r) % r)
           | (ji == r + ri % r)
           | (ji == 2 * r + ri // (r * r)))
    o_ref[...] = (
        jnp.dot(sel.astype(jnp.float32), tab_ref[...],
                preferred_element_type=jnp.float32)
        + lax.dot_general(f_ref[...], wfeat_ref[...],
                          (((0,), (0,)), ((), ())),
                          preferred_element_type=jnp.float32)
    )


def _build_a(tab, x2, wfeat):
    n = x2.shape[1]
    m = tab.shape[1]
    return pl.pallas_call(
        _build_a_kernel,
        out_shape=jax.ShapeDtypeStruct((n, m), jnp.float32),
    )(tab, x2, wfeat)


def _build_b_kernel(tab_ref, b_ref, o_ref):
    n = o_ref.shape[0]
    ra = tab_ref.shape[0]
    r = ra // 3
    ri = lax.broadcasted_iota(jnp.int32, (n, ra), 0)
    ji = lax.broadcasted_iota(jnp.int32, (n, ra), 1)
    sel = ((ji == ri // (r * r))
           | (ji == r + (ri // r) % r)
           | (ji == 2 * r + ri % r))
    o_ref[...] = (
        jnp.dot(sel.astype(jnp.float32), tab_ref[...],
                preferred_element_type=jnp.float32)
        + b_ref[...]
    )


def _build_b(tab, b, n):
    m = tab.shape[1]
    return pl.pallas_call(
        _build_b_kernel,
        out_shape=jax.ShapeDtypeStruct((n, m), jnp.float32),
    )(tab, b)


def _sc_gather(table, idx2d, epad):
    info = plsc.get_sparse_core_info()
    nw = info.num_cores * info.num_subcores
    jpair = 2 * (epad // 128 // nw)
    ja = max(8, int(round(jpair * CORE0_FRAC / 8.0)) * 8)
    jb = jpair - ja
    jmax = max(ja, jb)
    d = table.shape[1]
    mesh = plsc.VectorSubcoreMesh(core_axis_name="c", subcore_axis_name="s")

    nbuf = 4
    ahead = 2

    @functools.partial(
        pl.kernel,
        out_type=jax.ShapeDtypeStruct((epad, d), jnp.float32),
        mesh=mesh,
        scratch_types=[
            pltpu.VMEM((jmax, 128), jnp.int32),
            [pltpu.VMEM((128, d), jnp.float32) for _ in range(nbuf)],
            [pltpu.SemaphoreType.DMA for _ in range(nbuf)],
            [pltpu.SemaphoreType.DMA for _ in range(nbuf)],
        ],
    )
    def gather_k(table_hbm, idx_hbm, out_hbm, idx_v, bufs, gsems, wsems):
        cc = lax.axis_index("c")
        ss = lax.axis_index("s")

        def run(row0, cnt):
            pltpu.sync_copy(idx_hbm.at[pl.ds(row0, cnt)],
                            idx_v.at[pl.ds(0, cnt)])
            gcps = [None] * nbuf
            wcps = [None] * nbuf
            for j in range(min(ahead, cnt)):
                gcps[j % nbuf] = pltpu.async_copy(
                    table_hbm.at[idx_v.at[j]], bufs[j % nbuf], gsems[j % nbuf])
            for j in range(cnt):
                nxt = j + ahead
                if nxt < cnt:
                    if nxt >= nbuf:
                        wcps[nxt % nbuf].wait()
                    gcps[nxt % nbuf] = pltpu.async_copy(
                        table_hbm.at[idx_v.at[nxt]], bufs[nxt % nbuf],
                        gsems[nxt % nbuf])
                gcps[j % nbuf].wait()
                wcps[j % nbuf] = pltpu.async_copy(
                    bufs[j % nbuf], out_hbm.at[pl.ds((row0 + j) * 128, 128)],
                    wsems[j % nbuf])
            for j in range(max(0, cnt - nbuf), cnt):
                wcps[j % nbuf].wait()

        @pl.when(cc == 0)
        def _():
            run(ss * jpair, ja)

        @pl.when(cc == 1)
        def _():
            run(ss * jpair + ja, jb)

    return gather_k(table, idx2d)


def _edge_mlp_kernel(base_ref, segr_ref, g_ref, bq_ref,
                     w2_ref, b2_ref, w3_ref, b3_ref, w4_ref, b4_ref, o_ref):
    i = pl.program_id(0)

    @pl.when(i == 0)
    def _():
        o_ref[...] = jnp.zeros_like(o_ref)

    nsub = BLK // SUB
    segr = segr_ref[0]
    g = g_ref[...]
    iota_r = lax.broadcasted_iota(jnp.int32, (QWIN, SUB), 0)
    ohs = []
    parts = []
    for j in range(nsub):
        base = base_ref[i * nsub + j]
        segr_j = segr[:, j * SUB:(j + 1) * SUB]
        oh_tr = (segr_j - base == iota_r).astype(jnp.float32)
        ohs.append(oh_tr)
        bwin = bq_ref[pl.ds(base, QWIN), :]
        s = lax.dot_general(oh_tr, bwin, (((0,), (0,)), ((), ())),
                            preferred_element_type=jnp.float32)
        parts.append(_gelu(g[j * SUB:(j + 1) * SUB] + s))
    h = jnp.concatenate(parts, axis=0)
    h = _gelu(jnp.dot(h, w2_ref[...], preferred_element_type=jnp.float32)
              + b2_ref[...])
    h = _gelu(jnp.dot(h, w3_ref[...], preferred_element_type=jnp.float32)
              + b3_ref[...])
    kern = (jnp.dot(h, w4_ref[...], preferred_element_type=jnp.float32)
            + b4_ref[...])
    for j in range(nsub):
        base = base_ref[i * nsub + j]
        part = jnp.dot(ohs[j], kern[j * SUB:(j + 1) * SUB],
                       preferred_element_type=jnp.float32)
        o_ref[pl.ds(base, QWIN), :] = o_ref[pl.ds(base, QWIN), :] + part


def kernel(x, grid_coords, latent_grid, W1, b1, W2, b2, W3, b3, W4, b4,
           neighbors_index, segment_ids):
    T, B, C, D, H, W = x.shape
    input_grid = grid_coords[0].reshape(-1, 3)
    bmin = input_grid.min(axis=0)
    bmax = input_grid.max(axis=0)
    lat = bmin + (bmax - bmin) * latent_grid
    nq = lat.shape[0]
    emb = 3 * 2 * EMB_FREQS

    freqs = 1.0 / (MAX_POS ** (jnp.arange(EMB_FREQS, dtype=jnp.float32)
                               / EMB_FREQS))
    fvec = jnp.tile(jnp.concatenate([freqs, freqs]), 3).reshape(1, emb)
    halfpi = jnp.float32(1.5707963267948966)
    offv = jnp.tile(jnp.concatenate(
        [jnp.zeros((EMB_FREQS,), jnp.float32),
         jnp.full((EMB_FREQS,), halfpi, jnp.float32)]), 3).reshape(1, emb)
    rl = round(nq ** (1.0 / 3.0))
    t_in = jnp.concatenate([input_grid[::W * D, 0],
                            input_grid[:W * D:D, 1],
                            input_grid[:D, 2]]).reshape(3 * H, 1)
    t_lat = jnp.concatenate([lat[::rl * rl, 0],
                             lat[:rl * rl:rl, 1],
                             lat[:rl, 2]]).reshape(3 * rl, 1)
    tab_a = _axis_table(t_in, fvec, offv, W1[:emb])
    tab_b = _axis_table(t_lat, fvec, offv, W1[emb:2 * emb])
    a_tab = _build_a(tab_a, x.reshape(C, D * H * W), W1[2 * emb:])
    bq_tab = _build_b(tab_b, b1.reshape(1, -1), nq)

    e = neighbors_index.shape[0]
    nbr = neighbors_index.astype(jnp.int32)
    seg = segment_ids.astype(jnp.int32)
    nbr = ((nbr % D) * (H * W) + (nbr // (W * D)) * W + (nbr // D) % W)
    epad = ((e + 4095) // 4096) * 4096
    pad = epad - e
    nbr_p = jnp.concatenate([nbr, jnp.zeros((pad,), jnp.int32)])
    seg_p = jnp.concatenate([seg, jnp.full((pad,), -1, jnp.int32)])

    idx2d = nbr_p.reshape(-1, 128)
    starts = seg_p[::SUB]
    base_all = jnp.clip(starts - (starts % 8), 0, nq - QWIN).astype(jnp.int32)

    nch = NCHUNK
    ech = epad // nch
    nb = ech // BLK
    rows_per_ch = ech // 128
    sub_per_ch = ech // SUB
    outs = []
    for c in range(nch):
        g_c = _sc_gather(a_tab, idx2d[c * rows_per_ch:(c + 1) * rows_per_ch],
                         ech)
        seg_c = lax.slice(seg_p, (c * ech,), ((c + 1) * ech,))
        base_c = lax.slice(base_all, (c * sub_per_ch,),
                           ((c + 1) * sub_per_ch,))
        segr = seg_c.reshape(nb, 1, BLK)
        out_c = pl.pallas_call(
            _edge_mlp_kernel,
            grid=(nb,),
            in_specs=[
                pl.BlockSpec(memory_space=pltpu.SMEM),
                pl.BlockSpec((1, 1, BLK), lambda i: (i, 0, 0)),
                pl.BlockSpec((BLK, 128), lambda i: (i, 0)),
                pl.BlockSpec((nq, 128), lambda i: (0, 0)),
                pl.BlockSpec((128, 256), lambda i: (0, 0)),
                pl.BlockSpec((1, 256), lambda i: (0, 0)),
                pl.BlockSpec((256, 128), lambda i: (0, 0)),
                pl.BlockSpec((1, 128), lambda i: (0, 0)),
                pl.BlockSpec((128, 32), lambda i: (0, 0)),
                pl.BlockSpec((1, 32), lambda i: (0, 0)),
            ],
            out_specs=pl.BlockSpec((nq, 32), lambda i: (0, 0)),
            out_shape=jax.ShapeDtypeStruct((nq, 32), jnp.float32),
        )(base_c, segr, g_c, bq_tab, W2, b2.reshape(1, -1),
          W3, b3.reshape(1, -1), W4, b4.reshape(1, -1))
        outs.append(out_c)
    out_q = outs[0]
    for o in outs[1:]:
        out_q = out_q + o

    rl = round(nq ** (1.0 / 3.0))
    out = out_q.reshape(1, 1, rl, rl, rl, out_q.shape[-1])
    out = jnp.transpose(out, (1, 0, 5, 4, 2, 3))
    return out


# speedup vs baseline: 1.0227x; 1.0227x over previous
import functools

import jax
import jax.numpy as jnp
from jax import lax
from jax.experimental import pallas as pl
from jax.experimental.pallas import tpu as pltpu
from jax.experimental.pallas import tpu_sc as plsc

EMB_FREQS = 32
MAX_POS = 10000.0
BLK = 8192
SUB = 512
QWIN = 128
NCHUNK = 2
CORE0_FRAC = 0.72


def _sin_embed(p):
    k = jnp.arange(EMB_FREQS, dtype=jnp.float32)
    freqs = 1.0 / (MAX_POS ** (k / EMB_FREQS))
    ang = p[..., None] * freqs
    emb = jnp.concatenate([jnp.sin(ang), jnp.cos(ang)], axis=-1)
    return emb.reshape(p.shape[0], -1)


def _gelu(v):
    return 0.5 * v * (1.0 + lax.erf(v * 0.7071067811865476))


def _axis_table_kernel(t_ref, fvec_ref, offv_ref, w_ref, o_ref):
    ra = t_ref.shape[0]
    emb = fvec_ref.shape[1]
    r = ra // 3
    ang = t_ref[...] * fvec_ref[...] + offv_ref[...]
    e = jnp.sin(ang)
    grp = lax.broadcasted_iota(jnp.int32, (ra, emb), 0) // r
    colgrp = lax.broadcasted_iota(jnp.int32, (ra, emb), 1) // (emb // 3)
    e = jnp.where(grp == colgrp, e, 0.0)
    o_ref[...] = jnp.dot(e, w_ref[...], preferred_element_type=jnp.float32)


def _axis_table(t, fvec, offv, w):
    ra = t.shape[0]
    emb = fvec.shape[1]
    m = w.shape[1]
    return pl.pallas_call(
        _axis_table_kernel,
        out_shape=jax.ShapeDtypeStruct((ra, m), jnp.float32),
    )(t, fvec, offv, w)


def _build_a_kernel(tab_ref, f_ref, wfeat_ref, o_ref):
    n = o_ref.shape[0]
    ra = tab_ref.shape[0]
    r = ra // 3
    ri = lax.broadcasted_iota(jnp.int32, (n, ra), 0)
    ji = lax.broadcasted_iota(jnp.int32, (n, ra), 1)
    sel = ((ji == (ri // r) % r)
           | (ji == r + ri % r)
           | (ji == 2 * r + ri // (r * r)))
    o_ref[...] = (
        jnp.dot(sel.astype(jnp.float32), tab_ref[...],
                preferred_element_type=jnp.float32)
        + lax.dot_general(f_ref[...], wfeat_ref[...],
                          (((0,), (0,)), ((), ())),
                          preferred_element_type=jnp.float32)
    )


def _build_a(tab, x2, wfeat):
    n = x2.shape[1]
    m = tab.shape[1]
    return pl.pallas_call(
        _build_a_kernel,
        out_shape=jax.ShapeDtypeStruct((n, m), jnp.float32),
    )(tab, x2, wfeat)


def _build_b_kernel(tab_ref, b_ref, o_ref):
    n = o_ref.shape[0]
    ra = tab_ref.shape[0]
    r = ra // 3
    ri = lax.broadcasted_iota(jnp.int32, (n, ra), 0)
    ji = lax.broadcasted_iota(jnp.int32, (n, ra), 1)
    sel = ((ji == ri // (r * r))
           | (ji == r + (ri // r) % r)
           | (ji == 2 * r + ri % r))
    o_ref[...] = (
        jnp.dot(sel.astype(jnp.float32), tab_ref[...],
                preferred_element_type=jnp.float32)
        + b_ref[...]
    )


def _build_b(tab, b, n):
    m = tab.shape[1]
    return pl.pallas_call(
        _build_b_kernel,
        out_shape=jax.ShapeDtypeStruct((n, m), jnp.float32),
    )(tab, b)


def _sc_gather(table, idx2d, epad):
    info = plsc.get_sparse_core_info()
    nw = info.num_cores * info.num_subcores
    jpair = 2 * (epad // 128 // nw)
    ja = max(8, int(round(jpair * CORE0_FRAC / 8.0)) * 8)
    jb = jpair - ja
    jmax = max(ja, jb)
    d = table.shape[1]
    mesh = plsc.VectorSubcoreMesh(core_axis_name="c", subcore_axis_name="s")

    nbuf = 4
    ahead = 2

    @functools.partial(
        pl.kernel,
        out_type=jax.ShapeDtypeStruct((epad, d), jnp.float32),
        mesh=mesh,
        scratch_types=[
            pltpu.VMEM((jmax, 128), jnp.int32),
            [pltpu.VMEM((128, d), jnp.float32) for _ in range(nbuf)],
            [pltpu.SemaphoreType.DMA for _ in range(nbuf)],
            [pltpu.SemaphoreType.DMA for _ in range(nbuf)],
        ],
    )
    def gather_k(table_hbm, idx_hbm, out_hbm, idx_v, bufs, gsems, wsems):
        cc = lax.axis_index("c")
        ss = lax.axis_index("s")

        def run(row0, cnt):
            pltpu.sync_copy(idx_hbm.at[pl.ds(row0, cnt)],
                            idx_v.at[pl.ds(0, cnt)])
            gcps = [None] * nbuf
            wcps = [None] * nbuf
            for j in range(min(ahead, cnt)):
                gcps[j % nbuf] = pltpu.async_copy(
                    table_hbm.at[idx_v.at[j]], bufs[j % nbuf], gsems[j % nbuf])
            for j in range(cnt):
                nxt = j + ahead
                if nxt < cnt:
                    if nxt >= nbuf:
                        wcps[nxt % nbuf].wait()
                    gcps[nxt % nbuf] = pltpu.async_copy(
                        table_hbm.at[idx_v.at[nxt]], bufs[nxt % nbuf],
                        gsems[nxt % nbuf])
                gcps[j % nbuf].wait()
                wcps[j % nbuf] = pltpu.async_copy(
                    bufs[j % nbuf], out_hbm.at[pl.ds((row0 + j) * 128, 128)],
                    wsems[j % nbuf])
            for j in range(max(0, cnt - nbuf), cnt):
                wcps[j % nbuf].wait()

        @pl.when(cc == 0)
        def _():
            run(ss * jpair, ja)

        @pl.when(cc == 1)
        def _():
            run(ss * jpair + ja, jb)

    return gather_k(table, idx2d)


def _edge_mlp_kernel(base_ref, segr_ref, g_ref, bq_ref,
                     w2_ref, b2_ref, w3_ref, b3_ref, w4_ref, b4_ref, o_ref):
    i = pl.program_id(0)

    @pl.when(i == 0)
    def _():
        o_ref[...] = jnp.zeros_like(o_ref)

    nsub = BLK // SUB
    segr = segr_ref[0]
    g = g_ref[...]
    iota_r = lax.broadcasted_iota(jnp.int32, (QWIN, SUB), 0)
    ohs = []
    parts = []
    for j in range(nsub):
        base = base_ref[i * nsub + j]
        segr_j = segr[:, j * SUB:(j + 1) * SUB]
        oh_tr = (segr_j - base == iota_r).astype(jnp.float32)
        ohs.append(oh_tr)
        bwin = bq_ref[pl.ds(base, QWIN), :]
        s = lax.dot_general(oh_tr, bwin, (((0,), (0,)), ((), ())),
                            preferred_element_type=jnp.float32)
        parts.append(_gelu(g[j * SUB:(j + 1) * SUB] + s))
    h = jnp.concatenate(parts, axis=0)
    h = _gelu(jnp.dot(h, w2_ref[...], preferred_element_type=jnp.float32)
              + b2_ref[...])
    h = _gelu(jnp.dot(h, w3_ref[...], preferred_element_type=jnp.float32)
              + b3_ref[...])
    kern = (jnp.dot(h, w4_ref[...], preferred_element_type=jnp.float32)
            + b4_ref[...])
    for j in range(nsub):
        base = base_ref[i * nsub + j]
        part = jnp.dot(ohs[j], kern[j * SUB:(j + 1) * SUB],
                       preferred_element_type=jnp.float32)
        o_ref[pl.ds(base, QWIN), :] = o_ref[pl.ds(base, QWIN), :] + part


def kernel(x, grid_coords, latent_grid, W1, b1, W2, b2, W3, b3, W4, b4,
           neighbors_index, segment_ids):
    T, B, C, D, H, W = x.shape
    input_grid = grid_coords[0].reshape(-1, 3)
    bmin = input_grid.min(axis=0)
    bmax = input_grid.max(axis=0)
    lat = bmin + (bmax - bmin) * latent_grid
    nq = lat.shape[0]
    emb = 3 * 2 * EMB_FREQS

    freqs = 1.0 / (MAX_POS ** (jnp.arange(EMB_FREQS, dtype=jnp.float32)
                               / EMB_FREQS))
    fvec = jnp.tile(jnp.concatenate([freqs, freqs]), 3).reshape(1, emb)
    halfpi = jnp.float32(1.5707963267948966)
    offv = jnp.tile(jnp.concatenate(
        [jnp.zeros((EMB_FREQS,), jnp.float32),
         jnp.full((EMB_FREQS,), halfpi, jnp.float32)]), 3).reshape(1, emb)
    rl = round(nq ** (1.0 / 3.0))
    t_in = jnp.concatenate([input_grid[::W * D, 0],
                            input_grid[:W * D:D, 1],
                            input_grid[:D, 2]]).reshape(3 * H, 1)
    t_lat = jnp.concatenate([lat[::rl * rl, 0],
                             lat[:rl * rl:rl, 1],
                             lat[:rl, 2]]).reshape(3 * rl, 1)
    tab_a = _axis_table(t_in, fvec, offv, W1[:emb])
    tab_b = _axis_table(t_lat, fvec, offv, W1[emb:2 * emb])
    a_tab = _build_a(tab_a, x.reshape(C, D * H * W), W1[2 * emb:])
    bq_tab = _build_b(tab_b, b1.reshape(1, -1), nq)

    e = neighbors_index.shape[0]
    nbr = neighbors_index.astype(jnp.int32)
    seg = segment_ids.astype(jnp.int32)
    nbr = ((nbr % D) * (H * W) + (nbr // (W * D)) * W + (nbr // D) % W)
    epad = ((e + 4095) // 4096) * 4096
    pad = epad - e
    nbr_p = jnp.concatenate([nbr, jnp.zeros((pad,), jnp.int32)])
    seg_p = jnp.concatenate([seg, jnp.full((pad,), -1, jnp.int32)])

    idx2d = nbr_p.reshape(-1, 128)
    starts = seg_p[::SUB]
    base_all = jnp.clip(starts - (starts % 8), 0, nq - QWIN).astype(jnp.int32)

    nch = NCHUNK
    ech = epad // nch
    nb = ech // BLK
    rows_per_ch = ech // 128
    sub_per_ch = ech // SUB
    outs = []
    for c in range(nch):
        g_c = _sc_gather(a_tab, idx2d[c * rows_per_ch:(c + 1) * rows_per_ch],
                         ech)
        seg_c = lax.slice(seg_p, (c * ech,), ((c + 1) * ech,))
        base_c = lax.slice(base_all, (c * sub_per_ch,),
                           ((c + 1) * sub_per_ch,))
        segr = seg_c.reshape(nb, 1, BLK)
        out_c = pl.pallas_call(
            _edge_mlp_kernel,
            grid=(nb,),
            in_specs=[
                pl.BlockSpec(memory_space=pltpu.SMEM),
                pl.BlockSpec((1, 1, BLK), lambda i: (i, 0, 0)),
                pl.BlockSpec((BLK, 128), lambda i: (i, 0)),
                pl.BlockSpec((nq, 128), lambda i: (0, 0)),
                pl.BlockSpec((128, 256), lambda i: (0, 0)),
                pl.BlockSpec((1, 256), lambda i: (0, 0)),
                pl.BlockSpec((256, 128), lambda i: (0, 0)),
                pl.BlockSpec((1, 128), lambda i: (0, 0)),
                pl.BlockSpec((128, 32), lambda i: (0, 0)),
                pl.BlockSpec((1, 32), lambda i: (0, 0)),
            ],
            out_specs=pl.BlockSpec((nq, 32), lambda i: (0, 0)),
            out_shape=jax.ShapeDtypeStruct((nq, 32), jnp.float32),
        )(base_c, segr, g_c, bq_tab, W2, b2.reshape(1, -1),
          W3, b3.reshape(1, -1), W4, b4.reshape(1, -1))
        outs.append(out_c)
    out_q = outs[0]
    for o in outs[1:]:
        out_q = out_q + o

    rl = round(nq ** (1.0 / 3.0))
    out = out_q.reshape(1, 1, rl, rl, rl, out_q.shape[-1])
    out = jnp.transpose(out, (1, 0, 5, 4, 2, 3))
    return out
